# Initial kernel scaffold; baseline (speedup 1.0000x reference)
import jax
import jax.numpy as jnp
from jax.experimental import pallas as pl


def kernel(x, edge_index, cluster_id, cluster_index, W1, b1, g1, beta1, W2, b2, g2, beta2, W3, b3, g3, beta3, Wfc, bfc):
    raise NotImplementedError("write your pallas kernel here")


# speedup vs baseline: 5.9117x; 5.9117x over previous
import functools

import jax
import jax.numpy as jnp
from jax import lax
from jax.experimental import pallas as pl
from jax.experimental.pallas import tpu as pltpu
from jax.experimental.pallas import tpu_sc as plsc

N = 10000
E = 320000
D = 128
B = 4096
C = 64
NC = 2
NS = 16
NW = NC * NS
CH = 128
EPT = 10240
EPAD = EPT * NW
NCH = EPT // CH
NACC = 10240
RPT = NACC // NS
RB = 1000
EPS = 1e-5

_sc_cache = {}


def _sc_kernel(name, body, out_type, scratch_types):
    fn = _sc_cache.get(name)
    if fn is None:
        mesh = plsc.VectorSubcoreMesh(core_axis_name="c", subcore_axis_name="s",
                                      num_cores=NC, num_subcores=NS)
        fn = pl.kernel(body, out_type=out_type, mesh=mesh,
                       scratch_types=scratch_types)
        _sc_cache[name] = fn
    return fn


def _deg_body(dst_hbm, ones_hbm, z1_hbm, out_hbm, dstv, onesv, acc, sem):
    c = lax.axis_index("c")
    s = lax.axis_index("s")
    wid = c * NS + s
    pltpu.sync_copy(z1_hbm, acc.at[pl.ds(s * RPT, RPT)])
    pltpu.sync_copy(ones_hbm, onesv)
    plsc.subcore_barrier()

    def body(j, _):
        pltpu.sync_copy(dst_hbm.at[wid * NCH + j], dstv)
        pltpu.sync_copy(onesv, acc.at[dstv], add=True)
        return 0

    lax.fori_loop(0, NCH, body, 0)
    plsc.subcore_barrier()
    pltpu.sync_copy(acc.at[pl.ds(s * RPT, RPT)],
                    out_hbm.at[pl.ds(c * NACC + s * RPT, RPT)])


def _deg_call(*args):
    return _sc_kernel(
        "deg", _deg_body,
        jax.ShapeDtypeStruct((NC * NACC,), jnp.float32),
        [
            pltpu.VMEM((CH,), jnp.int32),
            pltpu.VMEM((CH,), jnp.float32),
            pltpu.VMEM_SHARED((NACC,), jnp.float32),
            pltpu.SemaphoreType.DMA,
        ],
    )(*args)


def _agg_body(h_hbm, src_hbm, dst_hbm, z2_hbm, out_hbm, srcv, dstv, rows, acc, sem):
    c = lax.axis_index("c")
    s = lax.axis_index("s")
    wid = c * NS + s
    pltpu.sync_copy(z2_hbm, acc.at[pl.ds(s * RPT, RPT)])
    pltpu.sync_copy(src_hbm.at[pl.ds(wid * NCH, NCH)], srcv)
    pltpu.sync_copy(dst_hbm.at[pl.ds(wid * NCH, NCH)], dstv)
    plsc.subcore_barrier()

    def body(j, _):
        pltpu.async_copy(h_hbm.at[srcv.at[j]], rows, sem).wait()
        pltpu.sync_copy(rows, acc.at[dstv.at[j]], add=True)
        return 0

    lax.fori_loop(0, NCH, body, 0)
    plsc.subcore_barrier()
    pltpu.sync_copy(acc.at[pl.ds(s * RPT, RPT)], out_hbm.at[c, pl.ds(s * RPT, RPT)])


def _agg_call(*args):
    return _sc_kernel(
        "agg", _agg_body,
        jax.ShapeDtypeStruct((NC, NACC, D), jnp.float32),
        [
            pltpu.VMEM((NCH, CH), jnp.int32),
            pltpu.VMEM((NCH, CH), jnp.int32),
            pltpu.VMEM((CH, D), jnp.float32),
            pltpu.VMEM_SHARED((NACC, D), jnp.float32),
            pltpu.SemaphoreType.DMA,
        ],
    )(*args)


def _cgather_body(x_hbm, idx_hbm, out_hbm, idxv, rows, sem):
    c = lax.axis_index("c")
    s = lax.axis_index("s")
    wid = c * NS + s
    base = pl.multiple_of(wid * (B // NW), 8)
    pltpu.sync_copy(idx_hbm.at[pl.ds(base, B // NW)], idxv)
    pltpu.async_copy(x_hbm.at[idxv], rows, sem).wait()
    pltpu.sync_copy(rows, out_hbm.at[pl.ds(base, B // NW)])


def _cgather_call(*args):
    return _sc_kernel(
        "cgather", _cgather_body,
        jax.ShapeDtypeStruct((B, D), jnp.float32),
        [
            pltpu.VMEM((B // NW,), jnp.int32),
            pltpu.VMEM((B // NW, D), jnp.float32),
            pltpu.SemaphoreType.DMA,
        ],
    )(*args)


def _prep_body(d0, d1, x, w, dis_out, hh_out):
    dis = lax.rsqrt(1.0 + d0[...] + d1[...])
    dis_out[...] = dis
    hh_out[...] = dis * jnp.dot(x[...], w[...], preferred_element_type=jnp.float32)


def _prep(d0, d1, x, w):
    grid = N // RB
    return pl.pallas_call(
        _prep_body,
        grid=(grid,),
        in_specs=[
            pl.BlockSpec((RB, 1), lambda i: (i, 0)),
            pl.BlockSpec((RB, 1), lambda i: (i, 0)),
            pl.BlockSpec((RB, D), lambda i: (i, 0)),
            pl.BlockSpec((D, D), lambda i: (0, 0)),
        ],
        out_specs=[
            pl.BlockSpec((RB, 1), lambda i: (i, 0)),
            pl.BlockSpec((RB, D), lambda i: (i, 0)),
        ],
        out_shape=[
            jax.ShapeDtypeStruct((N, 1), jnp.float32),
            jax.ShapeDtypeStruct((N, D), jnp.float32),
        ],
    )(d0, d1, x, w)


def _stats_body(s0, s1, hh, dis, b, h_out, st_out):
    i = pl.program_id(0)
    h = dis[...] * (s0[0] + s1[0] + hh[...]) + b[...]
    h_out[...] = h

    @pl.when(i == 0)
    def _():
        st_out[...] = jnp.zeros_like(st_out)

    st_out[0:1, :] += jnp.sum(h, axis=0, keepdims=True)
    st_out[1:2, :] += jnp.sum(h * h, axis=0, keepdims=True)


def _stats(sp, hh, dis, b):
    grid = N // RB
    return pl.pallas_call(
        _stats_body,
        grid=(grid,),
        in_specs=[
            pl.BlockSpec((1, RB, D), lambda i: (0, i, 0)),
            pl.BlockSpec((1, RB, D), lambda i: (1, i, 0)),
            pl.BlockSpec((RB, D), lambda i: (i, 0)),
            pl.BlockSpec((RB, 1), lambda i: (i, 0)),
            pl.BlockSpec((1, D), lambda i: (0, 0)),
        ],
        out_specs=[
            pl.BlockSpec((RB, D), lambda i: (i, 0)),
            pl.BlockSpec((8, D), lambda i: (0, 0)),
        ],
        out_shape=[
            jax.ShapeDtypeStruct((N, D), jnp.float32),
            jax.ShapeDtypeStruct((8, D), jnp.float32),
        ],
    )(sp, sp, hh, dis, b)


def _bnmm_body(h, st, g, beta, dis, w, out):
    mu = st[0:1, :] * (1.0 / N)
    var = st[1:2, :] * (1.0 / N) - mu * mu
    sc = lax.rsqrt(var + EPS) * g[...]
    xn = jnp.maximum((h[...] - mu) * sc + beta[...], 0.0)
    out[...] = dis[...] * jnp.dot(xn, w[...], preferred_element_type=jnp.float32)


def _bnmm(h, st, g, beta, dis, w):
    grid = N // RB
    return pl.pallas_call(
        _bnmm_body,
        grid=(grid,),
        in_specs=[
            pl.BlockSpec((RB, D), lambda i: (i, 0)),
            pl.BlockSpec((8, D), lambda i: (0, 0)),
            pl.BlockSpec((1, D), lambda i: (0, 0)),
            pl.BlockSpec((1, D), lambda i: (0, 0)),
            pl.BlockSpec((RB, 1), lambda i: (i, 0)),
            pl.BlockSpec((D, D), lambda i: (0, 0)),
        ],
        out_specs=pl.BlockSpec((RB, D), lambda i: (i, 0)),
        out_shape=jax.ShapeDtypeStruct((N, D), jnp.float32),
    )(h, st, g, beta, dis, w)


def _bnfinal_body(h, st, g, beta, out):
    mu = st[0:1, :] * (1.0 / N)
    var = st[1:2, :] * (1.0 / N) - mu * mu
    sc = lax.rsqrt(var + EPS) * g[...]
    out[...] = jnp.maximum((h[...] - mu) * sc + beta[...], 0.0)


def _bnfinal(h, st, g, beta):
    grid = N // RB
    return pl.pallas_call(
        _bnfinal_body,
        grid=(grid,),
        in_specs=[
            pl.BlockSpec((RB, D), lambda i: (i, 0)),
            pl.BlockSpec((8, D), lambda i: (0, 0)),
            pl.BlockSpec((1, D), lambda i: (0, 0)),
            pl.BlockSpec((1, D), lambda i: (0, 0)),
        ],
        out_specs=pl.BlockSpec((RB, D), lambda i: (i, 0)),
        out_shape=jax.ShapeDtypeStruct((N, D), jnp.float32),
    )(h, st, g, beta)


def _tail_body(cid, xc, wfc, bfc, out):
    cid_ = cid[...]
    xc_ = xc[...]
    colsum = jnp.sum(cid_, axis=0, keepdims=True)
    cidn = cid_ / colsum
    cf = lax.dot_general(cidn, xc_, (((0,), (0,)), ((), ())),
                         preferred_element_type=jnp.float32)
    rmax = jnp.max(cid_, axis=1, keepdims=True)
    io = lax.broadcasted_iota(jnp.int32, (B, C), 1)
    am = jnp.min(jnp.where(cid_ == rmax, io, C), axis=1, keepdims=True)
    oh = (io == am).astype(jnp.float32)
    x1 = jnp.dot(oh, cf, preferred_element_type=jnp.float32)
    wt = wfc[0:D, :]
    wb = wfc[D:2 * D, :]
    bias = bfc[...]
    out[0:B, :] = (jnp.dot(xc_, wt, preferred_element_type=jnp.float32)
                   + jnp.dot(x1, wb, preferred_element_type=jnp.float32) + bias)
    out[B:2 * B, :] = (jnp.dot(x1, wt, preferred_element_type=jnp.float32)
                       + jnp.dot(xc_, wb, preferred_element_type=jnp.float32) + bias)


def _tail(cid, xc, wfc, bfc):
    return pl.pallas_call(
        _tail_body,
        out_shape=jax.ShapeDtypeStruct((2 * B, 2 * D), jnp.float32),
    )(cid, xc, wfc, bfc)


def kernel(x, edge_index, cluster_id, cluster_index,
           W1, b1, g1, beta1, W2, b2, g2, beta2, W3, b3, g3, beta3, Wfc, bfc):
    pad = EPAD - E
    src = jnp.concatenate([edge_index[0], jnp.zeros((pad,), jnp.int32)])
    pad_dst = N + (jnp.arange(pad, dtype=jnp.int32) % (NACC - N))
    dst = jnp.concatenate([edge_index[1], pad_dst])
    srcp = src.reshape(EPAD // CH, CH)
    dstp = dst.reshape(EPAD // CH, CH)

    ones_ch = jnp.ones((CH,), jnp.float32)
    z1 = jnp.zeros((RPT,), jnp.float32)
    z2 = jnp.zeros((RPT, D), jnp.float32)

    degp = _deg_call(dstp, ones_ch, z1).reshape(NC, NACC)
    d0 = degp[0, :N].reshape(N, 1)
    d1 = degp[1, :N].reshape(N, 1)
    dis, hh = _prep(d0, d1, x, W1)

    layers = ((b1, g1, beta1, W2), (b2, g2, beta2, W3), (b3, g3, beta3, None))
    xo = None
    for b, g, beta, wnext in layers:
        sp = _agg_call(hh, srcp, dstp, z2)
        h, st = _stats(sp, hh, dis, b.reshape(1, D))
        if wnext is not None:
            hh = _bnmm(h, st, g.reshape(1, D), beta.reshape(1, D), dis, wnext)
        else:
            xo = _bnfinal(h, st, g.reshape(1, D), beta.reshape(1, D))

    xc = _cgather_call(xo, cluster_index)
    return _tail(cluster_id, xc, Wfc, bfc)


# speedup vs baseline: 23.8152x; 4.0285x over previous
import functools

import jax
import jax.numpy as jnp
from jax import lax
from jax.experimental import pallas as pl
from jax.experimental.pallas import tpu as pltpu
from jax.experimental.pallas import tpu_sc as plsc

N = 10000
E = 320000
D = 128
B = 4096
C = 64
NC = 2
NS = 16
NW = NC * NS
CH = 128
EPT = 10240
EPAD = EPT * NW
NCH = EPT // CH
NACC = 10240
RPT = NACC // NS
RB = 1000
EPS = 1e-5

_sc_cache = {}


def _sc_kernel(name, body, out_type, scratch_types):
    fn = _sc_cache.get(name)
    if fn is None:
        mesh = plsc.VectorSubcoreMesh(core_axis_name="c", subcore_axis_name="s",
                                      num_cores=NC, num_subcores=NS)
        fn = pl.kernel(body, out_type=out_type, mesh=mesh,
                       scratch_types=scratch_types)
        _sc_cache[name] = fn
    return fn


def _deg_body(dst_hbm, ones_hbm, z1_hbm, out_hbm, dstv, onesv, acc, sem):
    c = lax.axis_index("c")
    s = lax.axis_index("s")
    wid = c * NS + s
    pltpu.sync_copy(z1_hbm, acc.at[pl.ds(s * RPT, RPT)])
    pltpu.sync_copy(ones_hbm, onesv)
    plsc.subcore_barrier()

    def body(j, _):
        pltpu.sync_copy(dst_hbm.at[wid * NCH + j], dstv)
        pltpu.sync_copy(onesv, acc.at[dstv], add=True)
        return 0

    lax.fori_loop(0, NCH, body, 0)
    plsc.subcore_barrier()
    pltpu.sync_copy(acc.at[pl.ds(s * RPT, RPT)],
                    out_hbm.at[pl.ds(c * NACC + s * RPT, RPT)])


def _deg_call(*args):
    return _sc_kernel(
        "deg", _deg_body,
        jax.ShapeDtypeStruct((NC * NACC,), jnp.float32),
        [
            pltpu.VMEM((CH,), jnp.int32),
            pltpu.VMEM((CH,), jnp.float32),
            pltpu.VMEM_SHARED((NACC,), jnp.float32),
            pltpu.SemaphoreType.DMA,
        ],
    )(*args)


def _agg_body(h_hbm, src_hbm, dst_hbm, z2_hbm, out_hbm,
              srcv, dst0, dst1, rows0, rows1, acc, sem0, sem1, semd0, semd1):
    c = lax.axis_index("c")
    s = lax.axis_index("s")
    wid = c * NS + s
    pltpu.sync_copy(z2_hbm, acc.at[pl.ds(s * RPT, RPT)])
    pltpu.sync_copy(src_hbm.at[pl.ds(wid * NCH, NCH)], srcv)
    plsc.subcore_barrier()

    base = wid * NCH
    pltpu.async_copy(h_hbm.at[srcv.at[0]], rows0, sem0)
    pltpu.async_copy(dst_hbm.at[base], dst0, semd0)

    def body(t, _):
        j0 = 2 * t
        pltpu.async_copy(h_hbm.at[srcv.at[j0 + 1]], rows1, sem1)
        pltpu.async_copy(dst_hbm.at[base + j0 + 1], dst1, semd1)
        pltpu.make_async_copy(h_hbm.at[srcv.at[j0]], rows0, sem0).wait()
        pltpu.make_async_copy(dst_hbm.at[base + j0], dst0, semd0).wait()
        pltpu.sync_copy(rows0, acc.at[dst0], add=True)

        @pl.when(t + 1 < NCH // 2)
        def _():
            pltpu.async_copy(h_hbm.at[srcv.at[j0 + 2]], rows0, sem0)
            pltpu.async_copy(dst_hbm.at[base + j0 + 2], dst0, semd0)

        pltpu.make_async_copy(h_hbm.at[srcv.at[j0 + 1]], rows1, sem1).wait()
        pltpu.make_async_copy(dst_hbm.at[base + j0 + 1], dst1, semd1).wait()
        pltpu.sync_copy(rows1, acc.at[dst1], add=True)
        return 0

    lax.fori_loop(0, NCH // 2, body, 0)
    plsc.subcore_barrier()
    pltpu.sync_copy(acc.at[pl.ds(s * RPT, RPT)], out_hbm.at[c, pl.ds(s * RPT, RPT)])


def _agg_call(*args):
    return _sc_kernel(
        "agg", _agg_body,
        jax.ShapeDtypeStruct((NC, NACC, D), jnp.float32),
        [
            pltpu.VMEM((NCH, CH), jnp.int32),
            pltpu.VMEM((CH,), jnp.int32),
            pltpu.VMEM((CH,), jnp.int32),
            pltpu.VMEM((CH, D), jnp.float32),
            pltpu.VMEM((CH, D), jnp.float32),
            pltpu.VMEM_SHARED((NACC, D), jnp.float32),
            pltpu.SemaphoreType.DMA,
            pltpu.SemaphoreType.DMA,
            pltpu.SemaphoreType.DMA,
            pltpu.SemaphoreType.DMA,
        ],
    )(*args)


def _cgather_body(x_hbm, idx_hbm, out_hbm, idxv, rows, sem):
    c = lax.axis_index("c")
    s = lax.axis_index("s")
    wid = c * NS + s
    base = pl.multiple_of(wid * (B // NW), 8)
    pltpu.sync_copy(idx_hbm.at[pl.ds(base, B // NW)], idxv)
    pltpu.async_copy(x_hbm.at[idxv], rows, sem).wait()
    pltpu.sync_copy(rows, out_hbm.at[pl.ds(base, B // NW)])


def _cgather_call(*args):
    return _sc_kernel(
        "cgather", _cgather_body,
        jax.ShapeDtypeStruct((B, D), jnp.float32),
        [
            pltpu.VMEM((B // NW,), jnp.int32),
            pltpu.VMEM((B // NW, D), jnp.float32),
            pltpu.SemaphoreType.DMA,
        ],
    )(*args)


def _prep_body(d0, d1, x, w, dis_out, hh_out):
    dis = lax.rsqrt(1.0 + d0[...] + d1[...])
    dis_out[...] = dis
    hh_out[...] = dis * jnp.dot(x[...], w[...], preferred_element_type=jnp.float32)


def _prep(d0, d1, x, w):
    grid = N // RB
    return pl.pallas_call(
        _prep_body,
        grid=(grid,),
        in_specs=[
            pl.BlockSpec((RB, 1), lambda i: (i, 0)),
            pl.BlockSpec((RB, 1), lambda i: (i, 0)),
            pl.BlockSpec((RB, D), lambda i: (i, 0)),
            pl.BlockSpec((D, D), lambda i: (0, 0)),
        ],
        out_specs=[
            pl.BlockSpec((RB, 1), lambda i: (i, 0)),
            pl.BlockSpec((RB, D), lambda i: (i, 0)),
        ],
        out_shape=[
            jax.ShapeDtypeStruct((N, 1), jnp.float32),
            jax.ShapeDtypeStruct((N, D), jnp.float32),
        ],
    )(d0, d1, x, w)


def _stats_body(s0, s1, hh, dis, b, h_out, st_out):
    i = pl.program_id(0)
    h = dis[...] * (s0[0] + s1[0] + hh[...]) + b[...]
    h_out[...] = h

    @pl.when(i == 0)
    def _():
        st_out[...] = jnp.zeros_like(st_out)

    st_out[0:1, :] += jnp.sum(h, axis=0, keepdims=True)
    st_out[1:2, :] += jnp.sum(h * h, axis=0, keepdims=True)


def _stats(sp, hh, dis, b):
    grid = N // RB
    return pl.pallas_call(
        _stats_body,
        grid=(grid,),
        in_specs=[
            pl.BlockSpec((1, RB, D), lambda i: (0, i, 0)),
            pl.BlockSpec((1, RB, D), lambda i: (1, i, 0)),
            pl.BlockSpec((RB, D), lambda i: (i, 0)),
            pl.BlockSpec((RB, 1), lambda i: (i, 0)),
            pl.BlockSpec((1, D), lambda i: (0, 0)),
        ],
        out_specs=[
            pl.BlockSpec((RB, D), lambda i: (i, 0)),
            pl.BlockSpec((8, D), lambda i: (0, 0)),
        ],
        out_shape=[
            jax.ShapeDtypeStruct((N, D), jnp.float32),
            jax.ShapeDtypeStruct((8, D), jnp.float32),
        ],
    )(sp, sp, hh, dis, b)


def _bnmm_body(h, st, g, beta, dis, w, out):
    mu = st[0:1, :] * (1.0 / N)
    var = st[1:2, :] * (1.0 / N) - mu * mu
    sc = lax.rsqrt(var + EPS) * g[...]
    xn = jnp.maximum((h[...] - mu) * sc + beta[...], 0.0)
    out[...] = dis[...] * jnp.dot(xn, w[...], preferred_element_type=jnp.float32)


def _bnmm(h, st, g, beta, dis, w):
    grid = N // RB
    return pl.pallas_call(
        _bnmm_body,
        grid=(grid,),
        in_specs=[
            pl.BlockSpec((RB, D), lambda i: (i, 0)),
            pl.BlockSpec((8, D), lambda i: (0, 0)),
            pl.BlockSpec((1, D), lambda i: (0, 0)),
            pl.BlockSpec((1, D), lambda i: (0, 0)),
            pl.BlockSpec((RB, 1), lambda i: (i, 0)),
            pl.BlockSpec((D, D), lambda i: (0, 0)),
        ],
        out_specs=pl.BlockSpec((RB, D), lambda i: (i, 0)),
        out_shape=jax.ShapeDtypeStruct((N, D), jnp.float32),
    )(h, st, g, beta, dis, w)


def _bnfinal_body(h, st, g, beta, out):
    mu = st[0:1, :] * (1.0 / N)
    var = st[1:2, :] * (1.0 / N) - mu * mu
    sc = lax.rsqrt(var + EPS) * g[...]
    out[...] = jnp.maximum((h[...] - mu) * sc + beta[...], 0.0)


def _bnfinal(h, st, g, beta):
    grid = N // RB
    return pl.pallas_call(
        _bnfinal_body,
        grid=(grid,),
        in_specs=[
            pl.BlockSpec((RB, D), lambda i: (i, 0)),
            pl.BlockSpec((8, D), lambda i: (0, 0)),
            pl.BlockSpec((1, D), lambda i: (0, 0)),
            pl.BlockSpec((1, D), lambda i: (0, 0)),
        ],
        out_specs=pl.BlockSpec((RB, D), lambda i: (i, 0)),
        out_shape=jax.ShapeDtypeStruct((N, D), jnp.float32),
    )(h, st, g, beta)


def _tail_body(cid, xc, wfc, bfc, out):
    cid_ = cid[...]
    xc_ = xc[...]
    colsum = jnp.sum(cid_, axis=0, keepdims=True)
    cidn = cid_ / colsum
    cf = lax.dot_general(cidn, xc_, (((0,), (0,)), ((), ())),
                         preferred_element_type=jnp.float32)
    rmax = jnp.max(cid_, axis=1, keepdims=True)
    io = lax.broadcasted_iota(jnp.int32, (B, C), 1)
    am = jnp.min(jnp.where(cid_ == rmax, io, C), axis=1, keepdims=True)
    oh = (io == am).astype(jnp.float32)
    x1 = jnp.dot(oh, cf, preferred_element_type=jnp.float32)
    wt = wfc[0:D, :]
    wb = wfc[D:2 * D, :]
    bias = bfc[...]
    out[0:B, :] = (jnp.dot(xc_, wt, preferred_element_type=jnp.float32)
                   + jnp.dot(x1, wb, preferred_element_type=jnp.float32) + bias)
    out[B:2 * B, :] = (jnp.dot(x1, wt, preferred_element_type=jnp.float32)
                       + jnp.dot(xc_, wb, preferred_element_type=jnp.float32) + bias)


def _tail(cid, xc, wfc, bfc):
    return pl.pallas_call(
        _tail_body,
        out_shape=jax.ShapeDtypeStruct((2 * B, 2 * D), jnp.float32),
    )(cid, xc, wfc, bfc)


def kernel(x, edge_index, cluster_id, cluster_index,
           W1, b1, g1, beta1, W2, b2, g2, beta2, W3, b3, g3, beta3, Wfc, bfc):
    pad = EPAD - E
    pad_src = jnp.arange(pad, dtype=jnp.int32) % N
    src = jnp.concatenate([edge_index[0], pad_src])
    pad_dst = N + (jnp.arange(pad, dtype=jnp.int32) % (NACC - N))
    dst = jnp.concatenate([edge_index[1], pad_dst])
    srcp = src.reshape(EPAD // CH, CH)
    dstp = dst.reshape(EPAD // CH, CH)

    ones_ch = jnp.ones((CH,), jnp.float32)
    z1 = jnp.zeros((RPT,), jnp.float32)
    z2 = jnp.zeros((RPT, D), jnp.float32)

    degp = _deg_call(dstp, ones_ch, z1).reshape(NC, NACC)
    d0 = degp[0, :N].reshape(N, 1)
    d1 = degp[1, :N].reshape(N, 1)
    dis, hh = _prep(d0, d1, x, W1)

    layers = ((b1, g1, beta1, W2), (b2, g2, beta2, W3), (b3, g3, beta3, None))
    xo = None
    for b, g, beta, wnext in layers:
        sp = _agg_call(hh, srcp, dstp, z2)
        h, st = _stats(sp, hh, dis, b.reshape(1, D))
        if wnext is not None:
            hh = _bnmm(h, st, g.reshape(1, D), beta.reshape(1, D), dis, wnext)
        else:
            xo = _bnfinal(h, st, g.reshape(1, D), beta.reshape(1, D))

    xc = _cgather_call(xo, cluster_index)
    return _tail(cluster_id, xc, Wfc, bfc)
